# baseline (speedup 1.0000x reference)
import functools

import jax
import jax.numpy as jnp
from jax import lax
from jax.experimental import pallas as pl
from jax.experimental.pallas import tpu as pltpu

_B, _N, _S, _D, _K = 4, 8192, 2048, 64, 16
_TS = 256
_PAD = 128
_CHUNK = 128


def _topk_body(points_ref, new_points_ref, idx_ref, work_ref):
    b = pl.program_id(0)
    p = points_ref[0]
    q = new_points_ref[0]
    psq = (p[0] * p[0] + p[1] * p[1]) + p[2] * p[2]
    qsq = (q[0] * q[0] + q[1] * q[1]) + q[2] * q[2]
    qb = q.astype(jnp.bfloat16)
    dot = lax.dot_general(qb, p, (((0,), (0,)), ((), ())),
                          preferred_element_type=jnp.float32)
    work_ref[...] = (psq[None, :] + qsq[:, None]) - 2.0 * dot
    iota_f = lax.broadcasted_iota(jnp.int32, (_TS, _N), 1).astype(jnp.float32)
    base = b * _N
    for k in range(_K):
        w = work_ref[...]
        m = jnp.min(w, axis=1, keepdims=True)
        cand = jnp.where(w == m, iota_f, jnp.float32(_N))
        idxf = jnp.min(cand, axis=1)
        idx_ref[0, k, :] = idxf.astype(jnp.int32) + base
        work_ref[...] = jnp.where(iota_f == idxf[:, None],
                                  jnp.float32(jnp.inf), w)


def _topk_indices(points, new_points):
    return pl.pallas_call(
        _topk_body,
        grid=(_B, _S // _TS),
        in_specs=[
            pl.BlockSpec((1, 3, _N), lambda b, s: (b, 0, 0)),
            pl.BlockSpec((1, 3, _TS), lambda b, s: (b, 0, s)),
        ],
        out_specs=pl.BlockSpec((1, _K, _TS), lambda b, s: (b, 0, s)),
        out_shape=jax.ShapeDtypeStruct((_B, _K, _S), jnp.int32),
        scratch_shapes=[pltpu.VMEM((_TS, _N), jnp.float32)],
    )(points, new_points)


def _make_sc_gather(total):
    from jax.experimental.pallas import tpu_sc as plsc

    info = plsc.get_sparse_core_info()
    nc, ns = info.num_cores, info.num_subcores
    nw = nc * ns
    per_w = total // nw
    n_chunks = per_w // _CHUNK
    mesh = plsc.VectorSubcoreMesh(core_axis_name="c", subcore_axis_name="s")

    @functools.partial(
        pl.kernel,
        mesh=mesh,
        out_type=jax.ShapeDtypeStruct((total, _PAD), jnp.float32),
        scratch_types=[
            pltpu.VMEM((_CHUNK,), jnp.int32),
            pltpu.VMEM((_CHUNK, _PAD), jnp.float32),
            pltpu.SemaphoreType.DMA,
        ],
    )
    def sc_gather(table_hbm, idx_hbm, out_hbm, idx_v, rows_v, sem):
        wid = lax.axis_index("s") * nc + lax.axis_index("c")
        w_base = wid * per_w

        def chunk(c, _):
            start = pl.multiple_of(w_base + c * _CHUNK, _CHUNK)
            pltpu.sync_copy(idx_hbm.at[pl.ds(start, _CHUNK)], idx_v)
            pltpu.async_copy(table_hbm.at[idx_v], rows_v, sem).wait()
            pltpu.sync_copy(rows_v, out_hbm.at[pl.ds(start, _CHUNK)])
            return _

        lax.fori_loop(0, n_chunks, chunk, None)

    return sc_gather


def kernel(points, new_points, features):
    idx = _topk_indices(points, new_points)
    p_t = jnp.transpose(points, (0, 2, 1))
    f_t = jnp.transpose(features, (0, 2, 1))
    pad = jnp.zeros((_B, _N, _PAD - 3 - _D), jnp.float32)
    table = jnp.concatenate([p_t, f_t, pad], axis=-1).reshape(_B * _N, _PAD)
    q_t = jnp.transpose(new_points, (0, 2, 1))
    hb = _B // 2
    gather = _make_sc_gather(hb * _K * _S)
    idx2 = idx.reshape(2, hb * _K * _S)
    halves = []
    for h in range(2):
        g = gather(table, idx2[h]).reshape(hb, _K, _S, _PAD)
        coords = g[..., :3] - q_t[h * hb:(h + 1) * hb, None, :, :]
        halves.append(jnp.concatenate([coords, g[..., 3:3 + _D]], axis=-1))
    out = jnp.concatenate(halves, axis=0)
    return jnp.transpose(out, (0, 3, 1, 2))


# speedup vs baseline: 1.0079x; 1.0079x over previous
import functools

import jax
import jax.numpy as jnp
from jax import lax
from jax.experimental import pallas as pl
from jax.experimental.pallas import tpu as pltpu

_B, _N, _S, _D, _K = 4, 8192, 2048, 64, 16
_TS = 256
_PAD = 128
_CHUNK = 128


def _topk_body(points_ref, new_points_ref, idx_ref, work_ref):
    b = pl.program_id(0)
    p = points_ref[0]
    q = new_points_ref[0]
    psq = (p[0] * p[0] + p[2] * p[2]) + p[1] * p[1]
    qsq = (q[0] * q[0] + q[1] * q[1]) + q[2] * q[2]
    qb = q.astype(jnp.bfloat16)
    dot = lax.dot_general(qb, p, (((0,), (0,)), ((), ())),
                          preferred_element_type=jnp.float32)
    work_ref[...] = (psq[None, :] + qsq[:, None]) - 2.0 * dot
    iota_f = lax.broadcasted_iota(jnp.int32, (_TS, _N), 1).astype(jnp.float32)
    base = b * _N
    for k in range(_K):
        w = work_ref[...]
        m = jnp.min(w, axis=1, keepdims=True)
        cand = jnp.where(w == m, iota_f, jnp.float32(_N))
        idxf = jnp.min(cand, axis=1)
        idx_ref[0, k, :] = idxf.astype(jnp.int32) + base
        work_ref[...] = jnp.where(iota_f == idxf[:, None],
                                  jnp.float32(jnp.inf), w)


def _topk_indices(points, new_points):
    return pl.pallas_call(
        _topk_body,
        grid=(_B, _S // _TS),
        in_specs=[
            pl.BlockSpec((1, 3, _N), lambda b, s: (b, 0, 0)),
            pl.BlockSpec((1, 3, _TS), lambda b, s: (b, 0, s)),
        ],
        out_specs=pl.BlockSpec((1, _K, _TS), lambda b, s: (b, 0, s)),
        out_shape=jax.ShapeDtypeStruct((_B, _K, _S), jnp.int32),
        scratch_shapes=[pltpu.VMEM((_TS, _N), jnp.float32)],
    )(points, new_points)


def _make_sc_gather(total):
    from jax.experimental.pallas import tpu_sc as plsc

    info = plsc.get_sparse_core_info()
    nc, ns = info.num_cores, info.num_subcores
    nw = nc * ns
    per_w = total // nw
    n_chunks = per_w // _CHUNK
    mesh = plsc.VectorSubcoreMesh(core_axis_name="c", subcore_axis_name="s")

    @functools.partial(
        pl.kernel,
        mesh=mesh,
        out_type=jax.ShapeDtypeStruct((total, _PAD), jnp.float32),
        scratch_types=[
            pltpu.VMEM((_CHUNK,), jnp.int32),
            pltpu.VMEM((_CHUNK, _PAD), jnp.float32),
            pltpu.SemaphoreType.DMA,
        ],
    )
    def sc_gather(table_hbm, idx_hbm, out_hbm, idx_v, rows_v, sem):
        wid = lax.axis_index("s") * nc + lax.axis_index("c")
        w_base = wid * per_w

        def chunk(c, _):
            start = pl.multiple_of(w_base + c * _CHUNK, _CHUNK)
            pltpu.sync_copy(idx_hbm.at[pl.ds(start, _CHUNK)], idx_v)
            pltpu.async_copy(table_hbm.at[idx_v], rows_v, sem).wait()
            pltpu.sync_copy(rows_v, out_hbm.at[pl.ds(start, _CHUNK)])
            return _

        lax.fori_loop(0, n_chunks, chunk, None)

    return sc_gather


def kernel(points, new_points, features):
    idx = _topk_indices(points, new_points)
    p_t = jnp.transpose(points, (0, 2, 1))
    f_t = jnp.transpose(features, (0, 2, 1))
    pad = jnp.zeros((_B, _N, _PAD - 3 - _D), jnp.float32)
    table = jnp.concatenate([p_t, f_t, pad], axis=-1).reshape(_B * _N, _PAD)
    q_t = jnp.transpose(new_points, (0, 2, 1))
    g = _make_sc_gather(_B * _K * _S)(table, idx.reshape(-1))
    g = g.reshape(_B, _K, _S, _PAD)
    coords = g[..., :3] - q_t[:, None, :, :]
    out = jnp.concatenate([coords, g[..., 3:3 + _D]], axis=-1)
    return jnp.transpose(out, (0, 3, 1, 2))
